# Initial kernel scaffold; baseline (speedup 1.0000x reference)
import jax
import jax.numpy as jnp
from jax.experimental import pallas as pl


def kernel(x, weight, bias):
    raise NotImplementedError("write your pallas kernel here")


# speedup vs baseline: 1.5482x; 1.5482x over previous
import jax
import jax.numpy as jnp
from jax.experimental import pallas as pl
from jax.experimental.pallas import tpu as pltpu

DIM = 4096
N_EXPERTS = 64
TOPK = 8
N_GROUPS = 8
GROUP_SIZE = N_EXPERTS // N_GROUPS
TOPK_GROUPS = 4
ROUTE_SCALE = 2.5
N_TOK = 8192

BLOCK_T = 256
NEG = -1e30


def _gate_block(x_ref, wT_ref, b_ref, w_out_ref, idx_out_ref):
    x = x_ref[...]
    wT = wT_ref[...]
    logits = jnp.dot(x, wT, preferred_element_type=jnp.float32)
    orig = jax.nn.sigmoid(logits)
    s = orig + b_ref[...]
    B = s.shape[0]
    lane = jax.lax.broadcasted_iota(jnp.int32, (B, N_EXPERTS), 1)

    gs_cols = []
    for g in range(N_GROUPS):
        m = (lane // GROUP_SIZE) == g
        vg = jnp.where(m, s, NEG)
        m1 = jnp.max(vg, axis=1, keepdims=True)
        eq = vg == m1
        a1 = jnp.min(jnp.where(eq, lane, N_EXPERTS), axis=1, keepdims=True)
        m2 = jnp.max(jnp.where(lane == a1, NEG, vg), axis=1, keepdims=True)
        gs_cols.append(m1 + m2)
    gsc = jnp.concatenate(gs_cols, axis=1)

    glane = jax.lax.broadcasted_iota(jnp.int32, (B, N_GROUPS), 1)
    gv = gsc
    gsel = jnp.zeros((B, N_GROUPS), jnp.bool_)
    for _ in range(TOPK_GROUPS):
        m = jnp.max(gv, axis=1, keepdims=True)
        a = jnp.min(jnp.where(gv == m, glane, N_GROUPS), axis=1, keepdims=True)
        hit = glane == a
        gsel = gsel | hit
        gv = jnp.where(hit, NEG, gv)

    lane_sel = jnp.zeros((B, N_EXPERTS), jnp.bool_)
    for g in range(N_GROUPS):
        gm = (lane // GROUP_SIZE) == g
        lane_sel = lane_sel | (gm & gsel[:, g:g + 1])

    v = jnp.where(lane_sel, s, NEG)
    idx_cols = []
    w_cols = []
    for _ in range(TOPK):
        m = jnp.max(v, axis=1, keepdims=True)
        a = jnp.min(jnp.where(v == m, lane, N_EXPERTS), axis=1, keepdims=True)
        hit = lane == a
        idx_cols.append(a)
        w_cols.append(jnp.sum(jnp.where(hit, orig, 0.0), axis=1, keepdims=True))
        v = jnp.where(hit, NEG, v)
    idx = jnp.concatenate(idx_cols, axis=1)
    w = jnp.concatenate(w_cols, axis=1)
    w = (w / jnp.sum(w, axis=1, keepdims=True)) * ROUTE_SCALE

    w_out_ref[...] = w
    idx_out_ref[...] = idx


def kernel(x, weight, bias):
    n = x.shape[0]
    wT = weight.T
    b2 = bias.reshape(1, N_EXPERTS)
    grid = (n // BLOCK_T,)
    w_out, idx_out = pl.pallas_call(
        _gate_block,
        grid=grid,
        in_specs=[
            pl.BlockSpec((BLOCK_T, DIM), lambda i: (i, 0)),
            pl.BlockSpec((DIM, N_EXPERTS), lambda i: (0, 0)),
            pl.BlockSpec((1, N_EXPERTS), lambda i: (0, 0)),
        ],
        out_specs=[
            pl.BlockSpec((BLOCK_T, TOPK), lambda i: (i, 0)),
            pl.BlockSpec((BLOCK_T, TOPK), lambda i: (i, 0)),
        ],
        out_shape=[
            jax.ShapeDtypeStruct((n, TOPK), jnp.float32),
            jax.ShapeDtypeStruct((n, TOPK), jnp.int32),
        ],
        compiler_params=pltpu.CompilerParams(
            dimension_semantics=("arbitrary",),
        ),
    )(x, wT, b2)
    return w_out, idx_out


# speedup vs baseline: 4.0279x; 2.6018x over previous
import jax
import jax.numpy as jnp
from jax.experimental import pallas as pl
from jax.experimental.pallas import tpu as pltpu

DIM = 4096
N_EXPERTS = 64
TOPK = 8
N_GROUPS = 8
GROUP_SIZE = N_EXPERTS // N_GROUPS
TOPK_GROUPS = 4
ROUTE_SCALE = 2.5
N_TOK = 8192

BLOCK_T = 256
NEG = -1e30


def _gate_block(x_ref, wT_ref, b_ref, w_out_ref, idx_out_ref):
    x = x_ref[...]
    wT = wT_ref[...]
    logits = jnp.dot(x, wT, preferred_element_type=jnp.float32)
    lt = logits.T
    B = lt.shape[1]
    origT = jax.nn.sigmoid(lt)
    sT = origT + b_ref[...]

    g3 = sT.reshape(N_GROUPS, GROUP_SIZE, B)
    rid = jax.lax.broadcasted_iota(jnp.int32, (N_GROUPS, GROUP_SIZE, B), 1)
    m1 = jnp.max(g3, axis=1, keepdims=True)
    a1 = jnp.min(jnp.where(g3 == m1, rid, GROUP_SIZE), axis=1, keepdims=True)
    m2 = jnp.max(jnp.where(rid == a1, NEG, g3), axis=1, keepdims=True)
    gsc = (m1 + m2).reshape(N_GROUPS, B)

    grow = jax.lax.broadcasted_iota(jnp.int32, (N_GROUPS, B), 0)
    gsel = jnp.zeros((N_GROUPS, B), jnp.bool_)
    for _ in range(TOPK_GROUPS):
        m = jnp.max(gsc, axis=0, keepdims=True)
        a = jnp.min(jnp.where(gsc == m, grow, N_GROUPS), axis=0, keepdims=True)
        hit = grow == a
        gsel = gsel | hit
        gsc = jnp.where(hit, NEG, gsc)

    row_sel = jnp.broadcast_to(
        gsel.reshape(N_GROUPS, 1, B), (N_GROUPS, GROUP_SIZE, B)
    ).reshape(N_EXPERTS, B)
    v = jnp.where(row_sel, sT, NEG)

    rows = jax.lax.broadcasted_iota(jnp.int32, (N_EXPERTS, B), 0)
    idx_rows = []
    w_rows = []
    for _ in range(TOPK):
        m = jnp.max(v, axis=0, keepdims=True)
        a = jnp.min(jnp.where(v == m, rows, N_EXPERTS), axis=0, keepdims=True)
        hit = rows == a
        idx_rows.append(a)
        w_rows.append(jnp.sum(jnp.where(hit, origT, 0.0), axis=0, keepdims=True))
        v = jnp.where(hit, NEG, v)
    idxT = jnp.concatenate(idx_rows, axis=0)
    wT_r = jnp.concatenate(w_rows, axis=0)
    wT_r = (wT_r / jnp.sum(wT_r, axis=0, keepdims=True)) * ROUTE_SCALE

    w_out_ref[...] = wT_r.T
    idx_out_ref[...] = idxT.T


def kernel(x, weight, bias):
    n = x.shape[0]
    wT = weight.T
    b2 = bias.reshape(N_EXPERTS, 1)
    grid = (n // BLOCK_T,)
    w_out, idx_out = pl.pallas_call(
        _gate_block,
        grid=grid,
        in_specs=[
            pl.BlockSpec((BLOCK_T, DIM), lambda i: (i, 0)),
            pl.BlockSpec((DIM, N_EXPERTS), lambda i: (0, 0)),
            pl.BlockSpec((N_EXPERTS, 1), lambda i: (0, 0)),
        ],
        out_specs=[
            pl.BlockSpec((BLOCK_T, TOPK), lambda i: (i, 0)),
            pl.BlockSpec((BLOCK_T, TOPK), lambda i: (i, 0)),
        ],
        out_shape=[
            jax.ShapeDtypeStruct((n, TOPK), jnp.float32),
            jax.ShapeDtypeStruct((n, TOPK), jnp.int32),
        ],
        compiler_params=pltpu.CompilerParams(
            dimension_semantics=("arbitrary",),
        ),
    )(x, wT, b2)
    return w_out, idx_out


# speedup vs baseline: 4.7934x; 1.1900x over previous
import jax
import jax.numpy as jnp
from jax.experimental import pallas as pl
from jax.experimental.pallas import tpu as pltpu

DIM = 4096
N_EXPERTS = 64
TOPK = 8
N_GROUPS = 8
GROUP_SIZE = N_EXPERTS // N_GROUPS
TOPK_GROUPS = 4
ROUTE_SCALE = 2.5
N_TOK = 8192

BLOCK_T = 512
NEG = -1e30


def _gate_block(x_ref, wT_ref, b_ref, w_out_ref, idx_out_ref):
    x = x_ref[...]
    wT = wT_ref[...]
    logits = jnp.dot(x, wT, preferred_element_type=jnp.float32)
    lt = logits.T
    B = lt.shape[1]
    origT = jax.nn.sigmoid(lt)
    sT = origT + b_ref[...]

    g3 = sT.reshape(N_GROUPS, GROUP_SIZE, B)
    rid = jax.lax.broadcasted_iota(jnp.int32, (N_GROUPS, GROUP_SIZE, B), 1)
    m1 = jnp.max(g3, axis=1, keepdims=True)
    a1 = jnp.min(jnp.where(g3 == m1, rid, GROUP_SIZE), axis=1, keepdims=True)
    m2 = jnp.max(jnp.where(rid == a1, NEG, g3), axis=1, keepdims=True)
    gsc = (m1 + m2).reshape(N_GROUPS, B)

    grow = jax.lax.broadcasted_iota(jnp.int32, (N_GROUPS, B), 0)
    gsel = jnp.zeros((N_GROUPS, B), jnp.bool_)
    for _ in range(TOPK_GROUPS):
        m = jnp.max(gsc, axis=0, keepdims=True)
        a = jnp.min(jnp.where(gsc == m, grow, N_GROUPS), axis=0, keepdims=True)
        hit = grow == a
        gsel = gsel | hit
        gsc = jnp.where(hit, NEG, gsc)

    row_sel = jnp.broadcast_to(
        gsel.reshape(N_GROUPS, 1, B), (N_GROUPS, GROUP_SIZE, B)
    ).reshape(N_EXPERTS, B)
    v = jnp.where(row_sel, sT, NEG)

    rows = jax.lax.broadcasted_iota(jnp.int32, (N_EXPERTS, B), 0)
    idx_rows = []
    w_rows = []
    for _ in range(TOPK):
        m = jnp.max(v, axis=0, keepdims=True)
        a = jnp.min(jnp.where(v == m, rows, N_EXPERTS), axis=0, keepdims=True)
        hit = rows == a
        idx_rows.append(a)
        w_rows.append(jnp.sum(jnp.where(hit, origT, 0.0), axis=0, keepdims=True))
        v = jnp.where(hit, NEG, v)
    idxT = jnp.concatenate(idx_rows, axis=0)
    wT_r = jnp.concatenate(w_rows, axis=0)
    wT_r = (wT_r / jnp.sum(wT_r, axis=0, keepdims=True)) * ROUTE_SCALE

    w_out_ref[...] = wT_r.T
    idx_out_ref[...] = idxT.T


def kernel(x, weight, bias):
    n = x.shape[0]
    wT = weight.T
    b2 = bias.reshape(N_EXPERTS, 1)
    grid = (n // BLOCK_T,)
    w_out, idx_out = pl.pallas_call(
        _gate_block,
        grid=grid,
        in_specs=[
            pl.BlockSpec((BLOCK_T, DIM), lambda i: (i, 0)),
            pl.BlockSpec((DIM, N_EXPERTS), lambda i: (0, 0)),
            pl.BlockSpec((N_EXPERTS, 1), lambda i: (0, 0)),
        ],
        out_specs=[
            pl.BlockSpec((BLOCK_T, TOPK), lambda i: (i, 0)),
            pl.BlockSpec((BLOCK_T, TOPK), lambda i: (i, 0)),
        ],
        out_shape=[
            jax.ShapeDtypeStruct((n, TOPK), jnp.float32),
            jax.ShapeDtypeStruct((n, TOPK), jnp.int32),
        ],
        compiler_params=pltpu.CompilerParams(
            dimension_semantics=("arbitrary",),
        ),
    )(x, wT, b2)
    return w_out, idx_out


# speedup vs baseline: 5.0079x; 1.0447x over previous
import jax
import jax.numpy as jnp
from jax.experimental import pallas as pl
from jax.experimental.pallas import tpu as pltpu

DIM = 4096
N_EXPERTS = 64
TOPK = 8
N_GROUPS = 8
GROUP_SIZE = N_EXPERTS // N_GROUPS
TOPK_GROUPS = 4
ROUTE_SCALE = 2.5
N_TOK = 8192

BLOCK_T = 1024
NEG = -1e30


def _gate_block(x_ref, wT_ref, b_ref, w_out_ref, idx_out_ref):
    x = x_ref[...]
    wT = wT_ref[...]
    logits = jnp.dot(x, wT, preferred_element_type=jnp.float32)
    lt = logits.T
    B = lt.shape[1]
    origT = jax.nn.sigmoid(lt)
    sT = origT + b_ref[...]

    g3 = sT.reshape(N_GROUPS, GROUP_SIZE, B)
    rid = jax.lax.broadcasted_iota(jnp.int32, (N_GROUPS, GROUP_SIZE, B), 1)
    m1 = jnp.max(g3, axis=1, keepdims=True)
    a1 = jnp.min(jnp.where(g3 == m1, rid, GROUP_SIZE), axis=1, keepdims=True)
    m2 = jnp.max(jnp.where(rid == a1, NEG, g3), axis=1, keepdims=True)
    gsc = (m1 + m2).reshape(N_GROUPS, B)

    grow = jax.lax.broadcasted_iota(jnp.int32, (N_GROUPS, B), 0)
    gsel = jnp.zeros((N_GROUPS, B), jnp.bool_)
    for _ in range(TOPK_GROUPS):
        m = jnp.max(gsc, axis=0, keepdims=True)
        a = jnp.min(jnp.where(gsc == m, grow, N_GROUPS), axis=0, keepdims=True)
        hit = grow == a
        gsel = gsel | hit
        gsc = jnp.where(hit, NEG, gsc)

    row_sel = jnp.broadcast_to(
        gsel.reshape(N_GROUPS, 1, B), (N_GROUPS, GROUP_SIZE, B)
    ).reshape(N_EXPERTS, B)
    v = jnp.where(row_sel, sT, NEG)

    rows = jax.lax.broadcasted_iota(jnp.int32, (N_EXPERTS, B), 0)
    idx_rows = []
    w_rows = []
    for _ in range(TOPK):
        m = jnp.max(v, axis=0, keepdims=True)
        a = jnp.min(jnp.where(v == m, rows, N_EXPERTS), axis=0, keepdims=True)
        hit = rows == a
        idx_rows.append(a)
        w_rows.append(jnp.sum(jnp.where(hit, origT, 0.0), axis=0, keepdims=True))
        v = jnp.where(hit, NEG, v)
    idxT = jnp.concatenate(idx_rows, axis=0)
    wT_r = jnp.concatenate(w_rows, axis=0)
    wT_r = (wT_r / jnp.sum(wT_r, axis=0, keepdims=True)) * ROUTE_SCALE

    w_out_ref[...] = wT_r.T
    idx_out_ref[...] = idxT.T


def kernel(x, weight, bias):
    n = x.shape[0]
    wT = weight.T
    b2 = bias.reshape(N_EXPERTS, 1)
    grid = (n // BLOCK_T,)
    w_out, idx_out = pl.pallas_call(
        _gate_block,
        grid=grid,
        in_specs=[
            pl.BlockSpec((BLOCK_T, DIM), lambda i: (i, 0)),
            pl.BlockSpec((DIM, N_EXPERTS), lambda i: (0, 0)),
            pl.BlockSpec((N_EXPERTS, 1), lambda i: (0, 0)),
        ],
        out_specs=[
            pl.BlockSpec((BLOCK_T, TOPK), lambda i: (i, 0)),
            pl.BlockSpec((BLOCK_T, TOPK), lambda i: (i, 0)),
        ],
        out_shape=[
            jax.ShapeDtypeStruct((n, TOPK), jnp.float32),
            jax.ShapeDtypeStruct((n, TOPK), jnp.int32),
        ],
        compiler_params=pltpu.CompilerParams(
            dimension_semantics=("arbitrary",),
        ),
    )(x, wT, b2)
    return w_out, idx_out
